# Initial kernel scaffold; baseline (speedup 1.0000x reference)
import jax
import jax.numpy as jnp
from jax.experimental import pallas as pl


def kernel(x):
    raise NotImplementedError("write your pallas kernel here")


# speedup vs baseline: 16.8061x; 16.8061x over previous
import functools

import jax
import jax.numpy as jnp
from jax.experimental import pallas as pl

_K = 256


def _body(x_ref, o_ref):
    x = x_ref[...]
    rows = x.shape[0]
    b = jax.lax.bitcast_convert_type(x, jnp.uint32)
    key = jnp.where(b >> 31 == jnp.uint32(1), ~b, b | jnp.uint32(0x80000000))

    def step(i, prefix):
        bit = jnp.uint32(1) << (jnp.uint32(31) - jnp.uint32(i))
        cand = prefix | bit
        cnt = jnp.sum((key >= cand).astype(jnp.int32), axis=-1, keepdims=True)
        return jnp.where(cnt >= _K, cand, prefix)

    prefix = jnp.zeros((rows, 1), jnp.uint32)
    thresh = jax.lax.fori_loop(0, 32, step, prefix)
    o_ref[...] = jnp.where(key >= thresh, x, jnp.zeros_like(x))


@jax.jit
def kernel(x):
    return pl.pallas_call(
        _body,
        out_shape=jax.ShapeDtypeStruct(x.shape, x.dtype),
    )(x)
